# baseline (speedup 1.0000x reference)
import jax
import jax.numpy as jnp
from jax import lax
from jax.experimental import pallas as pl
from jax.experimental.pallas import tpu as pltpu
from jax.experimental.pallas import tpu_sc as plsc

DIM = 64
HIST = 200
PAD = 208
NCHUNK = PAD // 16
GCHUNK = 104


def _body(uidx_hbm, idx_hbm, ut_hbm, it_hbm, w1t_hbm, b1_hbm, w2_hbm, b2_hbm,
          out_hbm,
          idx_v, uidx_v, rows_v, urow_v, w1t_v, b1_v, w2_v, b2_v, a_v, sem):
    cid = lax.axis_index("c")
    sid = lax.axis_index("s")
    is_main = jnp.logical_and(cid == 0, sid == 0)

    @pl.when(is_main)
    def _():
        pltpu.sync_copy(idx_hbm, idx_v)
        pltpu.sync_copy(uidx_hbm, uidx_v)
        cp_a = pltpu.async_copy(
            it_hbm.at[idx_v.at[pl.ds(0, GCHUNK)]],
            rows_v.at[pl.ds(0, GCHUNK), :], sem)
        cp_b = pltpu.async_copy(
            it_hbm.at[idx_v.at[pl.ds(GCHUNK, GCHUNK)]],
            rows_v.at[pl.ds(GCHUNK, GCHUNK), :], sem)
        cp_u = pltpu.async_copy(ut_hbm.at[uidx_v], urow_v, sem)

        pltpu.sync_copy(w1t_hbm, w1t_v)
        pltpu.sync_copy(b1_hbm, b1_v)
        pltpu.sync_copy(w2_hbm, w2_v)
        pltpu.sync_copy(b2_hbm, b2_v)

        def fold_step(kb, gs):
            w2c = w2_v[pl.ds(kb * 16, 16)]
            for i in range(16):
                k = kb * 16 + i
                w2k = w2c[i]
                gs = tuple(
                    gs[c] + w1t_v[k, pl.ds(c * 16, 16)] * w2k
                    for c in range(8))
            return gs

        zeros = jnp.zeros((16,), jnp.float32)
        g = lax.fori_loop(0, 4, fold_step, (zeros,) * 8)

        lane = lax.iota(jnp.int32, 16)

        def _shuf(v, sh):
            return v.at[lane ^ sh].get(mode="promise_in_bounds")

        def hsum(v):
            for sh in (8, 4, 2, 1):
                v = v + _shuf(v, sh)
            return v

        def hmax(v):
            for sh in (8, 4, 2, 1):
                v = jnp.maximum(v, _shuf(v, sh))
            return v

        cp_u.wait()
        uacc = (urow_v[0, pl.ds(0, 16)] * g[0]
                + urow_v[0, pl.ds(16, 16)] * g[1]
                + urow_v[0, pl.ds(32, 16)] * g[2]
                + urow_v[0, pl.ds(48, 16)] * g[3])
        bacc = (b1_v[pl.ds(0, 16)] * w2_v[pl.ds(0, 16)]
                + b1_v[pl.ds(16, 16)] * w2_v[pl.ds(16, 16)]
                + b1_v[pl.ds(32, 16)] * w2_v[pl.ds(32, 16)]
                + b1_v[pl.ds(48, 16)] * w2_v[pl.ds(48, 16)])
        b2c = b2_v[pl.ds(0, 16)]
        const = hsum(uacc + bacc + b2c)

        cp_a.wait()
        cp_b.wait()

        g4, g5, g6, g7 = g[4], g[5], g[6], g[7]
        lane_is = [lane == i for i in range(16)]

        def chunk_step(c, carry):
            base = c * 16
            av = jnp.zeros((16,), jnp.float32)
            for i in range(16):
                r = (rows_v[base + i, pl.ds(0, 16)] * g4
                     + rows_v[base + i, pl.ds(16, 16)] * g5
                     + rows_v[base + i, pl.ds(32, 16)] * g6
                     + rows_v[base + i, pl.ds(48, 16)] * g7)
                av = jnp.where(lane_is[i], hsum(r), av)
            s = av + const
            s = jnp.where(s >= 0.0, s, 0.01 * s)
            a_v[pl.ds(base, 16)] = s
            return carry

        lax.fori_loop(0, NCHUNK, chunk_step, 0)

        tail_mask = lane < (HIST - (NCHUNK - 1) * 16)

        neg_big = jnp.full((16,), -jnp.inf, jnp.float32)
        mvec = neg_big
        for c in range(NCHUNK):
            chunk = a_v[pl.ds(c * 16, 16)]
            if c == NCHUNK - 1:
                chunk = jnp.where(tail_mask, chunk, neg_big)
            mvec = jnp.maximum(mvec, chunk)
        m = hmax(mvec)

        svec = jnp.zeros((16,), jnp.float32)
        for c in range(NCHUNK):
            chunk = a_v[pl.ds(c * 16, 16)]
            e = jnp.exp(chunk - m)
            if c == NCHUNK - 1:
                e = jnp.where(tail_mask, e, 0.0)
            a_v[pl.ds(c * 16, 16)] = e
            svec = svec + e
        inv = 1.0 / hsum(svec)

        for c in range(NCHUNK):
            a_v[pl.ds(c * 16, 16)] = a_v[pl.ds(c * 16, 16)] * inv

        pltpu.sync_copy(a_v.at[pl.ds(0, HIST)], out_hbm)


@jax.jit
def _attention(uidx8, idx_all, user_table, item_table, w1t, b1, w2f, b2p):
    run = pl.kernel(
        _body,
        mesh=plsc.VectorSubcoreMesh(core_axis_name="c", subcore_axis_name="s"),
        out_type=jax.ShapeDtypeStruct((HIST,), jnp.float32),
        compiler_params=pltpu.CompilerParams(use_tc_tiling_on_sc=False),
        scratch_types=[
            pltpu.VMEM((PAD,), jnp.int32),
            pltpu.VMEM((8,), jnp.int32),
            pltpu.VMEM((PAD, DIM), jnp.float32),
            pltpu.VMEM((8, DIM), jnp.float32),
            pltpu.VMEM((DIM, 2 * DIM), jnp.float32),
            pltpu.VMEM((DIM,), jnp.float32),
            pltpu.VMEM((DIM,), jnp.float32),
            pltpu.VMEM((16,), jnp.float32),
            pltpu.VMEM((PAD,), jnp.float32),
            pltpu.SemaphoreType.DMA,
        ],
    )
    return run(uidx8, idx_all, user_table, item_table, w1t, b1, w2f, b2p)


def kernel(user_indice, interacted_item_indices, user_table, item_table,
           W1, b1, W2, b2):
    idx_all = jnp.concatenate(
        [interacted_item_indices.astype(jnp.int32),
         jnp.zeros((PAD - HIST,), jnp.int32)])
    uidx8 = jnp.full((8,), user_indice, dtype=jnp.int32)
    w1t = W1.T
    w2f = W2.reshape(DIM)
    b2p = jnp.pad(b2, (0, 15))
    return _attention(uidx8, idx_all, user_table, item_table, w1t, b1, w2f,
                      b2p)


# speedup vs baseline: 1.5795x; 1.5795x over previous
import jax
import jax.numpy as jnp
from jax import lax
from jax.experimental import pallas as pl
from jax.experimental.pallas import tpu as pltpu
from jax.experimental.pallas import tpu_sc as plsc

DIM = 64
HIST = 200
PAD = 208
NCHUNK = PAD // 16


def _body(uidx_hbm, idx_hbm, ut_hbm, it_hbm, w1t_hbm, b1_hbm, w2_hbm, b2_hbm,
          out_hbm,
          idx_v, uidx_v, rows_v, urow_v, w1t_v, b1_v, w2_v, b2_v, a_v, sem,
          usem):
    cid = lax.axis_index("c")
    sid = lax.axis_index("s")
    is_main = jnp.logical_and(cid == 0, sid == 0)

    @pl.when(is_main)
    def _():
        pltpu.sync_copy(idx_hbm, idx_v)
        pltpu.sync_copy(uidx_hbm, uidx_v)

        uidx = uidx_v[pl.ds(0, 16)]
        pltpu.async_copy(ut_hbm.at[uidx[0]], urow_v.at[0], usem)

        for c in range(NCHUNK):
            idxc = idx_v[pl.ds(c * 16, 16)]
            for i in range(16):
                pltpu.async_copy(
                    it_hbm.at[idxc[i]], rows_v.at[c * 16 + i], sem)

        pltpu.sync_copy(w1t_hbm, w1t_v)
        pltpu.sync_copy(b1_hbm, b1_v)
        pltpu.sync_copy(w2_hbm, w2_v)
        pltpu.sync_copy(b2_hbm, b2_v)

        def fold_step(kb, gs):
            w2c = w2_v[pl.ds(kb * 16, 16)]
            for i in range(16):
                k = kb * 16 + i
                w2k = w2c[i]
                gs = tuple(
                    gs[c] + w1t_v[k, pl.ds(c * 16, 16)] * w2k
                    for c in range(8))
            return gs

        zeros = jnp.zeros((16,), jnp.float32)
        g = lax.fori_loop(0, 4, fold_step, (zeros,) * 8)

        lane = lax.iota(jnp.int32, 16)

        def _shuf(v, sh):
            return v.at[lane ^ sh].get(mode="promise_in_bounds")

        def hsum(v):
            for sh in (8, 4, 2, 1):
                v = v + _shuf(v, sh)
            return v

        def hmax(v):
            for sh in (8, 4, 2, 1):
                v = jnp.maximum(v, _shuf(v, sh))
            return v

        pltpu.make_async_copy(ut_hbm.at[0], urow_v.at[0], usem).wait()
        uacc = (urow_v[0, pl.ds(0, 16)] * g[0]
                + urow_v[0, pl.ds(16, 16)] * g[1]
                + urow_v[0, pl.ds(32, 16)] * g[2]
                + urow_v[0, pl.ds(48, 16)] * g[3])
        bacc = (b1_v[pl.ds(0, 16)] * w2_v[pl.ds(0, 16)]
                + b1_v[pl.ds(16, 16)] * w2_v[pl.ds(16, 16)]
                + b1_v[pl.ds(32, 16)] * w2_v[pl.ds(32, 16)]
                + b1_v[pl.ds(48, 16)] * w2_v[pl.ds(48, 16)])
        b2c = b2_v[pl.ds(0, 16)]
        const = hsum(uacc + bacc + b2c)

        def drain(l, carry):
            pltpu.make_async_copy(it_hbm.at[0], rows_v.at[0], sem).wait()
            return carry

        lax.fori_loop(0, PAD, drain, 0)

        g4, g5, g6, g7 = g[4], g[5], g[6], g[7]
        lane_is = [lane == i for i in range(16)]

        def chunk_step(c, carry):
            base = c * 16
            av = jnp.zeros((16,), jnp.float32)
            for i in range(16):
                r = (rows_v[base + i, pl.ds(0, 16)] * g4
                     + rows_v[base + i, pl.ds(16, 16)] * g5
                     + rows_v[base + i, pl.ds(32, 16)] * g6
                     + rows_v[base + i, pl.ds(48, 16)] * g7)
                av = jnp.where(lane_is[i], hsum(r), av)
            s = av + const
            s = jnp.where(s >= 0.0, s, 0.01 * s)
            a_v[pl.ds(base, 16)] = s
            return carry

        lax.fori_loop(0, NCHUNK, chunk_step, 0)

        tail_mask = lane < (HIST - (NCHUNK - 1) * 16)

        neg_big = jnp.full((16,), -jnp.inf, jnp.float32)
        mvec = neg_big
        for c in range(NCHUNK):
            chunk = a_v[pl.ds(c * 16, 16)]
            if c == NCHUNK - 1:
                chunk = jnp.where(tail_mask, chunk, neg_big)
            mvec = jnp.maximum(mvec, chunk)
        m = hmax(mvec)

        svec = jnp.zeros((16,), jnp.float32)
        for c in range(NCHUNK):
            chunk = a_v[pl.ds(c * 16, 16)]
            e = jnp.exp(chunk - m)
            if c == NCHUNK - 1:
                e = jnp.where(tail_mask, e, 0.0)
            a_v[pl.ds(c * 16, 16)] = e
            svec = svec + e
        inv = 1.0 / hsum(svec)

        for c in range(NCHUNK):
            a_v[pl.ds(c * 16, 16)] = a_v[pl.ds(c * 16, 16)] * inv

        pltpu.sync_copy(a_v.at[pl.ds(0, HIST)], out_hbm)


@jax.jit
def _attention(uidx16, idx_all, user_table, item_table, w1t, b1, w2f, b2p):
    run = pl.kernel(
        _body,
        mesh=plsc.VectorSubcoreMesh(core_axis_name="c", subcore_axis_name="s"),
        out_type=jax.ShapeDtypeStruct((HIST,), jnp.float32),
        compiler_params=pltpu.CompilerParams(use_tc_tiling_on_sc=True),
        scratch_types=[
            pltpu.VMEM((PAD,), jnp.int32),
            pltpu.VMEM((16,), jnp.int32),
            pltpu.VMEM((PAD, DIM), jnp.float32),
            pltpu.VMEM((1, DIM), jnp.float32),
            pltpu.VMEM((DIM, 2 * DIM), jnp.float32),
            pltpu.VMEM((DIM,), jnp.float32),
            pltpu.VMEM((DIM,), jnp.float32),
            pltpu.VMEM((16,), jnp.float32),
            pltpu.VMEM((PAD,), jnp.float32),
            pltpu.SemaphoreType.DMA,
            pltpu.SemaphoreType.DMA,
        ],
    )
    return run(uidx16, idx_all, user_table, item_table, w1t, b1, w2f, b2p)


def kernel(user_indice, interacted_item_indices, user_table, item_table,
           W1, b1, W2, b2):
    idx_all = jnp.concatenate(
        [interacted_item_indices.astype(jnp.int32),
         jnp.zeros((PAD - HIST,), jnp.int32)])
    uidx16 = jnp.full((16,), user_indice, dtype=jnp.int32)
    w1t = W1.T
    w2f = W2.reshape(DIM)
    b2p = jnp.pad(b2, (0, 15))
    return _attention(uidx16, idx_all, user_table, item_table, w1t, b1, w2f,
                      b2p)
